# baseline (speedup 1.0000x reference)
import functools

import jax
import jax.numpy as jnp
from jax import lax
from jax.experimental import pallas as pl
from jax.experimental.pallas import tpu as pltpu
from jax.experimental.pallas import tpu_sc as plsc

MAX_IDX = 62
DLEV = 8
NDIM = 768
VPAD = 64
LANES = 16


def _fuse_body(table_ref, w_ref, b_ref, out_ref):
    t = table_ref[...]
    w = w_ref[...]
    acc = lax.dot_general(t, w, (((1,), (1,)), ((), ())),
                          preferred_element_type=jnp.float32)
    z = acc + b_ref[...][None, :]
    out_ref[...] = 0.5 * z * (1.0 + lax.erf(z * (2.0 ** -0.5)))


def _fused_table(table, w, b):
    tpad = jnp.zeros((VPAD, DLEV), jnp.float32).at[:table.shape[0]].set(table)
    return pl.pallas_call(
        _fuse_body,
        out_shape=jax.ShapeDtypeStruct((VPAD, NDIM), jnp.float32),
    )(tpad, w, b)


def _make_gather(n_rows):
    info = plsc.get_sparse_core_info()
    nw = info.num_cores * info.num_subcores
    chunk = 16
    nbuf = 8
    assert n_rows % (nw * chunk) == 0
    rows_per_w = n_rows // nw
    n_chunks = rows_per_w // chunk
    mesh = plsc.VectorSubcoreMesh(core_axis_name="c", subcore_axis_name="s")

    @functools.partial(
        pl.kernel,
        out_type=jax.ShapeDtypeStruct((n_rows, NDIM), jnp.float32),
        mesh=mesh,
        scratch_types=[
            pltpu.VMEM((rows_per_w,), jnp.int32),
            pltpu.VMEM((nbuf, chunk, NDIM), jnp.float32),
            pltpu.SemaphoreType.DMA((nbuf,)),
            pltpu.SemaphoreType.DMA((nbuf,)),
        ],
    )
    def gather(fused_hbm, idx_hbm, out_hbm, idx_v, rows_v, sem_g, sem_w):
        wid = lax.axis_index("s") * info.num_cores + lax.axis_index("c")
        w_base = wid * rows_per_w
        row_off = wid * VPAD

        pltpu.sync_copy(idx_hbm.at[pl.ds(w_base, rows_per_w)], idx_v)

        def bias(j, carry):
            sl = pl.ds(j * LANES, LANES)
            idx_v[sl] = idx_v[sl] + row_off
            return carry

        lax.fori_loop(0, rows_per_w // LANES, bias, 0)

        def g_desc(i, p):
            return pltpu.make_async_copy(
                fused_hbm.at[idx_v.at[pl.ds(i * chunk, chunk)]],
                rows_v.at[p], sem_g.at[p])

        def w_desc(i, p):
            return pltpu.make_async_copy(
                rows_v.at[p], out_hbm.at[pl.ds(w_base + i * chunk, chunk)],
                sem_w.at[p])

        for i in range(nbuf - 1):
            g_desc(i, i).start()

        def body(i, carry):
            p = lax.rem(i, nbuf)
            g_desc(i, p).wait()
            w_desc(i, p).start()
            nxt = i + nbuf - 1
            q = lax.rem(nxt, nbuf)

            @pl.when(i >= 1)
            def _():
                w_desc(i - 1, q).wait()

            @pl.when(nxt < n_chunks)
            def _():
                g_desc(nxt, q).start()

            return carry

        lax.fori_loop(0, n_chunks, body, 0)
        w_desc(n_chunks - 1, lax.rem(n_chunks - 1, nbuf)).wait()

    return gather


def _tc_gather_body(idx_ref, fused_ref, out_ref):
    idxb = idx_ref[0, 0, :]
    iota = lax.broadcasted_iota(jnp.int32, (idxb.shape[0], VPAD), 1)
    oh = (idxb[:, None] == iota).astype(jnp.float32)
    out_ref[...] = lax.dot_general(
        oh, fused_ref[...], (((1,), (0,)), ((), ())),
        preferred_element_type=jnp.float32)


def _tc_gather(fused, idx):
    rb = 1024
    n = idx.shape[0]
    g = n // rb
    idx3 = idx.reshape(g, 1, rb)
    return pl.pallas_call(
        _tc_gather_body,
        grid=(g,),
        in_specs=[
            pl.BlockSpec((1, 1, rb), lambda i: (i, 0, 0)),
            pl.BlockSpec((VPAD, NDIM), lambda i: (0, 0)),
        ],
        out_specs=pl.BlockSpec((rb, NDIM), lambda i: (i, 0)),
        out_shape=jax.ShapeDtypeStruct((n, NDIM), jnp.float32),
    )(idx3, fused)


def kernel(x, table, W, b):
    bsz, seq = x.shape
    info = plsc.get_sparse_core_info()
    nw = info.num_cores * info.num_subcores
    fused = _fused_table(table, W, b)
    idx = x.reshape(-1).astype(jnp.int32)
    half = idx.shape[0] // 2
    fused_rep = jnp.tile(fused, (nw, 1))
    fused_rep = jnp.tile(fused, (nw, 1))
    out = _make_gather(idx.shape[0])(fused_rep, idx)
    return out.reshape(bsz, seq, NDIM)


# speedup vs baseline: 1.1475x; 1.1475x over previous
import functools

import jax
import jax.numpy as jnp
from jax import lax
from jax.experimental import pallas as pl
from jax.experimental.pallas import tpu as pltpu
from jax.experimental.pallas import tpu_sc as plsc

MAX_IDX = 62
DLEV = 8
NDIM = 768
VPAD = 64
LANES = 16


def _fuse_body(table_ref, w_ref, b_ref, out_ref):
    t = table_ref[...]
    w = w_ref[...]
    acc = lax.dot_general(t, w, (((1,), (1,)), ((), ())),
                          preferred_element_type=jnp.float32)
    z = acc + b_ref[...][None, :]
    out_ref[...] = 0.5 * z * (1.0 + lax.erf(z * (2.0 ** -0.5)))


def _fused_table(table, w, b):
    tpad = jnp.zeros((VPAD, DLEV), jnp.float32).at[:table.shape[0]].set(table)
    return pl.pallas_call(
        _fuse_body,
        out_shape=jax.ShapeDtypeStruct((VPAD, NDIM), jnp.float32),
    )(tpad, w, b)


def _make_gather(n_rows):
    info = plsc.get_sparse_core_info()
    nw = info.num_cores * info.num_subcores
    chunk = 32
    nbuf = 4
    assert n_rows % (nw * chunk) == 0
    rows_per_w = n_rows // nw
    n_chunks = rows_per_w // chunk
    mesh = plsc.VectorSubcoreMesh(core_axis_name="c", subcore_axis_name="s")

    @functools.partial(
        pl.kernel,
        out_type=jax.ShapeDtypeStruct((n_rows, NDIM), jnp.float32),
        mesh=mesh,
        scratch_types=[
            pltpu.VMEM_SHARED((VPAD, NDIM), jnp.float32),
            pltpu.VMEM((rows_per_w,), jnp.int32),
            pltpu.VMEM((nbuf, chunk, NDIM), jnp.float32),
            pltpu.SemaphoreType.DMA,
            pltpu.SemaphoreType.DMA((nbuf,)),
        ],
    )
    def gather(fused_hbm, idx_hbm, out_hbm, table_sp, idx_v, rows_v,
               sem_b, sem_w):
        sid = lax.axis_index("s")
        wid = sid * info.num_cores + lax.axis_index("c")
        w_base = wid * rows_per_w

        @pl.when(sid == 0)
        def _():
            pltpu.sync_copy(fused_hbm, table_sp)

        pltpu.sync_copy(idx_hbm.at[pl.ds(w_base, rows_per_w)], idx_v)
        plsc.subcore_barrier()

        def w_desc(i, p):
            return pltpu.make_async_copy(
                rows_v.at[p], out_hbm.at[pl.ds(w_base + i * chunk, chunk)],
                sem_w.at[p])

        def body(i, carry):
            p = lax.rem(i, nbuf)

            @pl.when(i >= nbuf)
            def _():
                w_desc(i - nbuf, p).wait()

            descs = []
            for s in range(chunk // LANES):
                xv = idx_v[pl.ds(i * chunk + s * LANES, LANES)]
                for r in range(LANES):
                    d = pltpu.make_async_copy(
                        table_sp.at[xv[r]], rows_v.at[p, s * LANES + r], sem_b)
                    d.start()
                    descs.append(d)
            for d in descs:
                d.wait()

            w_desc(i, p).start()
            return carry

        lax.fori_loop(0, n_chunks, body, 0)

        def drain(i, carry):
            w_desc(i, lax.rem(i, nbuf)).wait()
            return carry

        lax.fori_loop(n_chunks - nbuf, n_chunks, drain, 0)

    return gather


def _tc_gather_body(idx_ref, fused_ref, out_ref):
    idxb = idx_ref[0, 0, :]
    iota = lax.broadcasted_iota(jnp.int32, (idxb.shape[0], VPAD), 1)
    oh = (idxb[:, None] == iota).astype(jnp.float32)
    out_ref[...] = lax.dot_general(
        oh, fused_ref[...], (((1,), (0,)), ((), ())),
        preferred_element_type=jnp.float32)


def _tc_gather(fused, idx):
    rb = 1024
    n = idx.shape[0]
    g = n // rb
    idx3 = idx.reshape(g, 1, rb)
    return pl.pallas_call(
        _tc_gather_body,
        grid=(g,),
        in_specs=[
            pl.BlockSpec((1, 1, rb), lambda i: (i, 0, 0)),
            pl.BlockSpec((VPAD, NDIM), lambda i: (0, 0)),
        ],
        out_specs=pl.BlockSpec((rb, NDIM), lambda i: (i, 0)),
        out_shape=jax.ShapeDtypeStruct((n, NDIM), jnp.float32),
    )(idx3, fused)


def kernel(x, table, W, b):
    bsz, seq = x.shape
    info = plsc.get_sparse_core_info()
    nw = info.num_cores * info.num_subcores
    fused = _fused_table(table, W, b)
    idx = x.reshape(-1).astype(jnp.int32)
    half = idx.shape[0] // 2
    fused_rep = jnp.tile(fused, (nw, 1))
    out = _make_gather(idx.shape[0])(fused, idx)
    return out.reshape(bsz, seq, NDIM)
